# Initial kernel scaffold; baseline (speedup 1.0000x reference)
import jax
import jax.numpy as jnp
from jax.experimental import pallas as pl


def kernel(indices, table):
    raise NotImplementedError("write your pallas kernel here")


# speedup vs baseline: 1.1025x; 1.1025x over previous
import functools

import jax
import jax.numpy as jnp
from jax import lax
from jax.experimental import pallas as pl
from jax.experimental.pallas import tpu as pltpu
from jax.experimental.pallas import tpu_sc as plsc

_D = 32
_N = 16384 * 50
_NC, _NS = 2, 16
_NW = _NC * _NS
_PER_W = _N // _NW
_C = 1600
_NCHUNK = _PER_W // _C

_mesh = plsc.VectorSubcoreMesh(
    core_axis_name="c", subcore_axis_name="s",
    num_cores=_NC, num_subcores=_NS,
)


@functools.partial(
    pl.kernel,
    out_type=jax.ShapeDtypeStruct((_N, _D), jnp.float32),
    mesh=_mesh,
    compiler_params=pltpu.CompilerParams(use_tc_tiling_on_sc=False),
    scratch_types=[
        pltpu.VMEM((_C,), jnp.int32),
        pltpu.VMEM((_C,), jnp.int32),
        pltpu.VMEM((_C, _D), jnp.float32),
        pltpu.VMEM((_C, _D), jnp.float32),
        pltpu.SemaphoreType.DMA,
    ],
)
def _embed_gather(idx_hbm, table_hbm, out_hbm, idx_a, idx_b, rows_a, rows_b, gsem):
    wid = lax.axis_index("s") * _NC + lax.axis_index("c")
    base = wid * _PER_W
    for g in range(_NCHUNK):
        idx_v = idx_a if g % 2 == 0 else idx_b
        rows_v = rows_a if g % 2 == 0 else rows_b
        off = base + g * _C
        pltpu.sync_copy(idx_hbm.at[pl.ds(off, _C)], idx_v)
        pltpu.async_copy(table_hbm.at[idx_v], rows_v, gsem).wait()
        pltpu.sync_copy(rows_v, out_hbm.at[pl.ds(off, _C)])


def kernel(indices, table):
    idx_flat = indices.reshape(-1).astype(jnp.int32)
    out = _embed_gather(idx_flat, table)
    return out.reshape(indices.shape[0], indices.shape[1], _D)
